# baseline (speedup 1.0000x reference)
import functools

import jax
import jax.numpy as jnp
from jax import lax
from jax.experimental import pallas as pl
from jax.experimental.pallas import tpu as pltpu
from jax.experimental.pallas import tpu_sc as plsc

_B, _S, _D = 4096, 200, 128
_N = _B * _S
_EPS = 1e-12
_V_TYPE, _V_DPE = 14, 25
_TDP = 384

_NC, _NS = 2, 16
_NW = _NC * _NS
_NSLAB = 4
_NSL = _N // _NSLAB
_RPW = _NSL // _NW
_CHUNK = 128
_NCHUNK = _RPW // _CHUNK
_NBUF = 4


def _sc_gather(ids, w_in):
    mesh = plsc.VectorSubcoreMesh(core_axis_name="c", subcore_axis_name="s")

    @functools.partial(
        pl.kernel,
        out_type=jax.ShapeDtypeStruct((_NSL, _D), jnp.float32),
        mesh=mesh,
        scratch_types=[pltpu.VMEM((_RPW,), jnp.int32)]
        + [pltpu.VMEM((_CHUNK, _D), jnp.float32) for _ in range(_NBUF)]
        + [pltpu.SemaphoreType.DMA for _ in range(2 * _NBUF)],
    )
    def k(ids_hbm, win_hbm, out_hbm, idx_v, *rest):
        bufs = rest[:_NBUF]
        sg = rest[_NBUF:2 * _NBUF]
        so = rest[2 * _NBUF:3 * _NBUF]
        wid = lax.axis_index("s") * _NC + lax.axis_index("c")
        base = wid * _RPW

        pltpu.sync_copy(ids_hbm.at[pl.ds(base, _RPW)], idx_v)

        def gather_desc(c, p):
            return pltpu.make_async_copy(
                win_hbm.at[idx_v.at[pl.ds(c * _CHUNK, _CHUNK)]], bufs[p], sg[p])

        def out_desc(c, p):
            return pltpu.make_async_copy(
                bufs[p], out_hbm.at[pl.ds(base + c * _CHUNK, _CHUNK)], so[p])

        for p in range(_NBUF - 1):
            gather_desc(p, p).start()

        for c in range(_NCHUNK):
            p = c % _NBUF
            gather_desc(c, p).wait()
            out_desc(c, p).start()
            if c >= 1:
                out_desc(c - 1, (c - 1) % _NBUF).wait()
            if c + _NBUF - 1 < _NCHUNK:
                gather_desc(c + _NBUF - 1, (c + _NBUF - 1) % _NBUF).start()
        out_desc(_NCHUNK - 1, (_NCHUNK - 1) % _NBUF).wait()

    return k(ids, w_in)


_RBLK = 4096
_SBLKS = _NSL // _RBLK


def _tc_add_layernorm_slab(x_slab, ct_slab, w_td16, gamma, beta, y_prev, slab):
    def body(*refs):
        x_ref, ct_ref, w_ref, g_ref, b_ref = refs[:5]
        o_ref = refs[-1]
        ids = ct_ref[...]
        oh = (ids == lax.broadcasted_iota(jnp.int32, (_RBLK, _TDP), 1))
        td = jnp.dot(oh.astype(jnp.bfloat16), w_ref[...],
                     preferred_element_type=jnp.float32)
        xv = x_ref[...] + td
        mean = jnp.mean(xv, axis=1, keepdims=True)
        xc = xv - mean
        var = jnp.mean(xc * xc, axis=1, keepdims=True)
        o_ref[...] = xc * lax.rsqrt(var + _EPS) * g_ref[...] + b_ref[...]

    in_specs = [
        pl.BlockSpec((_RBLK, _D), lambda i: (i, 0)),
        pl.BlockSpec((_RBLK, 1), lambda i: (i, 0)),
        pl.BlockSpec((_TDP, _D), lambda i: (0, 0)),
        pl.BlockSpec((1, _D), lambda i: (0, 0)),
        pl.BlockSpec((1, _D), lambda i: (0, 0)),
    ]
    args = [x_slab, ct_slab, w_td16, gamma.reshape(1, _D), beta.reshape(1, _D)]
    aliases = {}
    if y_prev is not None:
        in_specs.append(pl.BlockSpec(memory_space=pl.ANY))
        args.append(y_prev)
        aliases = {5: 0}
    return pl.pallas_call(
        body,
        grid=(_SBLKS,),
        in_specs=in_specs,
        out_specs=pl.BlockSpec((_RBLK, _D), lambda i, _s=slab: (i + _s * _SBLKS, 0)),
        out_shape=jax.ShapeDtypeStruct((_N, _D), jnp.float32),
        input_output_aliases=aliases,
    )(*args)


def kernel(input_ids, type_ids, dpe_ids, W_input, W_type, W_dpe, gamma, beta):
    ids = input_ids.reshape(_N).astype(jnp.int32)
    ct = (type_ids.reshape(_N).astype(jnp.int32) * _V_DPE
          + dpe_ids.reshape(_N).astype(jnp.int32)).reshape(_N, 1)
    w_td = (W_type[:, None, :] + W_dpe[None, :, :]).reshape(_V_TYPE * _V_DPE, _D)
    w_td16 = jnp.zeros((_TDP, _D), jnp.bfloat16).at[:_V_TYPE * _V_DPE].set(
        w_td.astype(jnp.bfloat16))
    y = None
    for s in range(_NSLAB):
        sum_s = _sc_gather(ids[s * _NSL:(s + 1) * _NSL], W_input)
        y = _tc_add_layernorm_slab(sum_s, ct[s * _NSL:(s + 1) * _NSL],
                                   w_td16, gamma, beta, y, s)
    return y.reshape(_B, _S, _D)


# speedup vs baseline: 1.2086x; 1.2086x over previous
import functools

import jax
import jax.numpy as jnp
from jax import lax
from jax.experimental import pallas as pl
from jax.experimental.pallas import tpu as pltpu
from jax.experimental.pallas import tpu_sc as plsc

_B, _S, _D = 4096, 200, 128
_N = _B * _S
_EPS = 1e-12
_V_TYPE, _V_DPE = 14, 25

_NC, _NS = 2, 16
_NW = _NC * _NS
_NSLAB = 8
_NSL = _N // _NSLAB
_RPW = _NSL // _NW
_CHUNK = 128
_NCHUNK = _RPW // _CHUNK
_NBUF = 6


def _sc_gather_sum(ids, ct, w_in, w_td):
    mesh = plsc.VectorSubcoreMesh(core_axis_name="c", subcore_axis_name="s")

    @functools.partial(
        pl.kernel,
        out_type=jax.ShapeDtypeStruct((_NSL, _D), jnp.float32),
        mesh=mesh,
        scratch_types=[
            pltpu.VMEM((_RPW,), jnp.int32),
            pltpu.VMEM((_RPW,), jnp.int32),
        ]
        + [pltpu.VMEM((_CHUNK, _D), jnp.float32) for _ in range(_NBUF)]
        + [pltpu.SemaphoreType.DMA for _ in range(3 * _NBUF)],
    )
    def k(ids_hbm, ct_hbm, win_hbm, wtd_hbm, out_hbm, idx_v, ct_v, *rest):
        bufs = rest[:_NBUF]
        sg = rest[_NBUF:2 * _NBUF]
        sa = rest[2 * _NBUF:3 * _NBUF]
        so = rest[3 * _NBUF:4 * _NBUF]
        wid = lax.axis_index("s") * _NC + lax.axis_index("c")
        base = wid * _RPW

        pltpu.sync_copy(ids_hbm.at[pl.ds(base, _RPW)], idx_v)
        pltpu.sync_copy(ct_hbm.at[pl.ds(base, _RPW)], ct_v)

        def gather_desc(c, p):
            return pltpu.make_async_copy(
                win_hbm.at[idx_v.at[pl.ds(c * _CHUNK, _CHUNK)]], bufs[p], sg[p])

        def add_desc(c, p):
            return pltpu.make_async_copy(
                wtd_hbm.at[ct_v.at[pl.ds(c * _CHUNK, _CHUNK)]], bufs[p], sa[p])

        def out_desc(c, p):
            return pltpu.make_async_copy(
                bufs[p], out_hbm.at[pl.ds(base + c * _CHUNK, _CHUNK)], so[p])

        def issue_add(c, p):
            pltpu.async_copy(
                wtd_hbm.at[ct_v.at[pl.ds(c * _CHUNK, _CHUNK)]], bufs[p], sa[p],
                add=True)

        for p in range(3):
            gather_desc(p, p).start()

        for c in range(_NCHUNK):
            p = c % _NBUF
            gather_desc(c, p).wait()
            issue_add(c, p)
            if c >= 1:
                add_desc(c - 1, (c - 1) % _NBUF).wait()
                out_desc(c - 1, (c - 1) % _NBUF).start()
            if c >= 3:
                out_desc(c - 3, (c - 3) % _NBUF).wait()
            if c + 3 < _NCHUNK:
                gather_desc(c + 3, (c + 3) % _NBUF).start()
        c_last = _NCHUNK - 1
        add_desc(c_last, c_last % _NBUF).wait()
        out_desc(c_last, c_last % _NBUF).start()
        for c in (c_last - 2, c_last - 1, c_last):
            out_desc(c, c % _NBUF).wait()

    return k(ids, ct, w_in, w_td)


_RBLK = 4096
_SBLKS = _NSL // _RBLK


def _tc_layernorm_slab(x_slab, gamma, beta, y_prev, slab):
    def body(*refs):
        x_ref, g_ref, b_ref = refs[0], refs[1], refs[2]
        o_ref = refs[-1]
        xv = x_ref[...]
        mean = jnp.mean(xv, axis=1, keepdims=True)
        xc = xv - mean
        var = jnp.mean(xc * xc, axis=1, keepdims=True)
        o_ref[...] = xc * lax.rsqrt(var + _EPS) * g_ref[...] + b_ref[...]

    in_specs = [
        pl.BlockSpec((_RBLK, _D), lambda i: (i, 0)),
        pl.BlockSpec((1, _D), lambda i: (0, 0)),
        pl.BlockSpec((1, _D), lambda i: (0, 0)),
    ]
    args = [x_slab, gamma.reshape(1, _D), beta.reshape(1, _D)]
    aliases = {}
    if y_prev is not None:
        in_specs.append(pl.BlockSpec(memory_space=pl.ANY))
        args.append(y_prev)
        aliases = {3: 0}
    return pl.pallas_call(
        body,
        grid=(_SBLKS,),
        in_specs=in_specs,
        out_specs=pl.BlockSpec((_RBLK, _D), lambda i, _s=slab: (i + _s * _SBLKS, 0)),
        out_shape=jax.ShapeDtypeStruct((_N, _D), jnp.float32),
        input_output_aliases=aliases,
    )(*args)


def kernel(input_ids, type_ids, dpe_ids, W_input, W_type, W_dpe, gamma, beta):
    ids = input_ids.reshape(_N).astype(jnp.int32)
    ct = (type_ids.reshape(_N).astype(jnp.int32) * _V_DPE
          + dpe_ids.reshape(_N).astype(jnp.int32))
    w_td = (W_type[:, None, :] + W_dpe[None, :, :]).reshape(_V_TYPE * _V_DPE, _D)
    y = None
    for s in range(_NSLAB):
        sum_s = _sc_gather_sum(ids[s * _NSL:(s + 1) * _NSL],
                               ct[s * _NSL:(s + 1) * _NSL], W_input, w_td)
        y = _tc_layernorm_slab(sum_s, gamma, beta, y, s)
    return y.reshape(_B, _S, _D)
